# baseline (device time: 279331 ns/iter reference)
import jax
import jax.numpy as jnp
from jax import lax
from jax.experimental import pallas as pl
from jax.experimental.pallas import tpu as pltpu

N_DEV = 4


def kernel(A, B):
    m_per, k = A.shape
    k2, n = B.shape
    assert k == k2
    half = m_per // 2
    q = m_per // 4

    def body(a_ref, b_ref, out_ref, c_own, c_top, c_bot, a_top_recv,
             a_bot_recv, a_send_sems, a_recv_sems, c_send_sems, c_recv_sems,
             copy_sems):
        my = lax.axis_index("i")
        left = (my + N_DEV - 1) % N_DEV
        right = (my + 1) % N_DEV
        diag = (my + 2) % N_DEV

        with jax.named_scope("barrier"):
            barrier_sem = pltpu.get_barrier_semaphore()
            for nbr in (left, right):
                pl.semaphore_signal(
                    barrier_sem, inc=1,
                    device_id=(nbr,), device_id_type=pl.DeviceIdType.MESH,
                )
            pl.semaphore_wait(barrier_sem, 2)

        a_top_send = pltpu.make_async_remote_copy(
            src_ref=a_ref.at[pl.ds(0, half), :],
            dst_ref=a_top_recv,
            send_sem=a_send_sems.at[0],
            recv_sem=a_recv_sems.at[0],
            device_id=(left,),
            device_id_type=pl.DeviceIdType.MESH,
        )
        a_top_send.start()
        a_bot_send = pltpu.make_async_remote_copy(
            src_ref=a_ref.at[pl.ds(half, half), :],
            dst_ref=a_bot_recv,
            send_sem=a_send_sems.at[1],
            recv_sem=a_recv_sems.at[1],
            device_id=(right,),
            device_id_type=pl.DeviceIdType.MESH,
        )
        a_bot_send.start()

        local_copies = []

        def ship(buf_at, grow, slot, peer):
            cp = pltpu.make_async_copy(
                buf_at, out_ref.at[pl.ds(grow, q), :], copy_sems.at[slot]
            )
            cp.start()
            local_copies.append(cp)
            pltpu.make_async_remote_copy(
                src_ref=buf_at,
                dst_ref=out_ref.at[pl.ds(grow, q), :],
                send_sem=c_send_sems.at[slot],
                recv_sem=c_recv_sems.at[slot],
                device_id=(peer,),
                device_id_type=pl.DeviceIdType.MESH,
            ).start()

        for qi, peer, slot in ((0, right, 0), (2, left, 2),
                               (1, right, 1), (3, left, 3)):
            with jax.named_scope(f"own#q={qi}"):
                c_own[pl.ds(qi * q, q), :] = jnp.dot(
                    a_ref[pl.ds(qi * q, q), :], b_ref[:, :],
                    preferred_element_type=jnp.float32,
                )
                ship(c_own.at[pl.ds(qi * q, q), :], my * m_per + qi * q,
                     slot, peer)

        with jax.named_scope("wait_a"):
            a_top_send.wait_recv()
            a_bot_send.wait_recv()
        for qi in range(2):
            with jax.named_scope(f"gap#q={qi}"):
                c_top[pl.ds(qi * q, q), :] = jnp.dot(
                    a_top_recv[pl.ds(qi * q, q), :], b_ref[:, :],
                    preferred_element_type=jnp.float32,
                )
                ship(c_top.at[pl.ds(qi * q, q), :], right * m_per + qi * q,
                     4 + qi, left)
                c_bot[pl.ds(qi * q, q), :] = jnp.dot(
                    a_bot_recv[pl.ds(qi * q, q), :], b_ref[:, :],
                    preferred_element_type=jnp.float32,
                )
                ship(c_bot.at[pl.ds(qi * q, q), :],
                     left * m_per + half + qi * q, 6 + qi, right)

        with jax.named_scope("drain_send"):
            a_top_send.wait_send()
            a_bot_send.wait_send()
            for slot in range(8):
                pltpu.make_async_remote_copy(
                    src_ref=c_own.at[pl.ds(0, q), :],
                    dst_ref=out_ref.at[pl.ds(0, q), :],
                    send_sem=c_send_sems.at[slot],
                    recv_sem=c_recv_sems.at[slot],
                    device_id=(right,),
                    device_id_type=pl.DeviceIdType.MESH,
                ).wait_send()
        with jax.named_scope("drain_copy"):
            for cp in local_copies:
                cp.wait()
        inbound = (
            (0, left * m_per + 0 * q),
            (1, left * m_per + 1 * q),
            (2, right * m_per + 2 * q),
            (3, right * m_per + 3 * q),
            (4, diag * m_per + 0 * q),
            (5, diag * m_per + 1 * q),
            (6, diag * m_per + half + 0 * q),
            (7, diag * m_per + half + 1 * q),
        )
        for slot, grow in inbound:
            with jax.named_scope(f"drain_recv#slot={slot}"):
                pltpu.make_async_remote_copy(
                    src_ref=c_own.at[pl.ds(0, q), :],
                    dst_ref=out_ref.at[pl.ds(grow, q), :],
                    send_sem=c_send_sems.at[slot],
                    recv_sem=c_recv_sems.at[slot],
                    device_id=(left,),
                    device_id_type=pl.DeviceIdType.MESH,
                ).wait_recv()

    out_shape = jax.ShapeDtypeStruct((N_DEV * m_per, n), jnp.float32)
    return pl.pallas_call(
        body,
        out_shape=out_shape,
        in_specs=[
            pl.BlockSpec(memory_space=pltpu.VMEM),
            pl.BlockSpec(memory_space=pltpu.VMEM),
        ],
        out_specs=pl.BlockSpec(memory_space=pltpu.MemorySpace.HBM),
        scratch_shapes=[
            pltpu.VMEM((m_per, n), jnp.float32),
            pltpu.VMEM((half, n), jnp.float32),
            pltpu.VMEM((half, n), jnp.float32),
            pltpu.VMEM((half, k), jnp.float32),
            pltpu.VMEM((half, k), jnp.float32),
            pltpu.SemaphoreType.DMA((2,)),
            pltpu.SemaphoreType.DMA((2,)),
            pltpu.SemaphoreType.DMA((8,)),
            pltpu.SemaphoreType.DMA((8,)),
            pltpu.SemaphoreType.DMA((8,)),
        ],
        compiler_params=pltpu.CompilerParams(
            collective_id=0, vmem_limit_bytes=100 * 1024 * 1024
        ),
    )(A, B)


# device time: 279306 ns/iter; 1.0001x vs baseline; 1.0001x over previous
import jax
import jax.numpy as jnp
from jax import lax
from jax.experimental import pallas as pl
from jax.experimental.pallas import tpu as pltpu

N_DEV = 4


def kernel(A, B):
    m_per, k = A.shape
    k2, n = B.shape
    assert k == k2
    half = m_per // 2
    q = m_per // 4

    def body(a_ref, b_ref, out_ref, c_own, c_top, c_bot, a_top_recv,
             a_bot_recv, a_send_sems, a_recv_sems, c_send_sems, c_recv_sems,
             copy_sems):
        my = lax.axis_index("i")
        left = (my + N_DEV - 1) % N_DEV
        right = (my + 1) % N_DEV
        diag = (my + 2) % N_DEV

        with jax.named_scope("barrier"):
            barrier_sem = pltpu.get_barrier_semaphore()
            for nbr in (left, right):
                pl.semaphore_signal(
                    barrier_sem, inc=1,
                    device_id=(nbr,), device_id_type=pl.DeviceIdType.MESH,
                )
            pl.semaphore_wait(barrier_sem, 2)

        a_top_send = pltpu.make_async_remote_copy(
            src_ref=a_ref.at[pl.ds(0, half), :],
            dst_ref=a_top_recv,
            send_sem=a_send_sems.at[0],
            recv_sem=a_recv_sems.at[0],
            device_id=(left,),
            device_id_type=pl.DeviceIdType.MESH,
        )
        a_top_send.start()
        a_bot_send = pltpu.make_async_remote_copy(
            src_ref=a_ref.at[pl.ds(half, half), :],
            dst_ref=a_bot_recv,
            send_sem=a_send_sems.at[1],
            recv_sem=a_recv_sems.at[1],
            device_id=(right,),
            device_id_type=pl.DeviceIdType.MESH,
        )
        a_bot_send.start()

        local_copies = []
        b_bf = b_ref[:, :].astype(jnp.bfloat16)

        def ship(buf_at, grow, slot, peer):
            cp = pltpu.make_async_copy(
                buf_at, out_ref.at[pl.ds(grow, q), :], copy_sems.at[slot]
            )
            cp.start()
            local_copies.append(cp)
            pltpu.make_async_remote_copy(
                src_ref=buf_at,
                dst_ref=out_ref.at[pl.ds(grow, q), :],
                send_sem=c_send_sems.at[slot],
                recv_sem=c_recv_sems.at[slot],
                device_id=(peer,),
                device_id_type=pl.DeviceIdType.MESH,
            ).start()

        for qi, peer, slot in ((0, right, 0), (2, left, 2),
                               (1, right, 1), (3, left, 3)):
            with jax.named_scope(f"own#q={qi}"):
                c_own[pl.ds(qi * q, q), :] = jnp.dot(
                    a_ref[pl.ds(qi * q, q), :].astype(jnp.bfloat16), b_bf,
                    preferred_element_type=jnp.float32,
                )
                ship(c_own.at[pl.ds(qi * q, q), :], my * m_per + qi * q,
                     slot, peer)

        with jax.named_scope("wait_a"):
            a_top_send.wait_recv()
            a_bot_send.wait_recv()
        for qi in range(2):
            with jax.named_scope(f"gap#q={qi}"):
                c_top[pl.ds(qi * q, q), :] = jnp.dot(
                    a_top_recv[pl.ds(qi * q, q), :].astype(jnp.bfloat16), b_bf,
                    preferred_element_type=jnp.float32,
                )
                ship(c_top.at[pl.ds(qi * q, q), :], right * m_per + qi * q,
                     4 + qi, left)
                c_bot[pl.ds(qi * q, q), :] = jnp.dot(
                    a_bot_recv[pl.ds(qi * q, q), :].astype(jnp.bfloat16), b_bf,
                    preferred_element_type=jnp.float32,
                )
                ship(c_bot.at[pl.ds(qi * q, q), :],
                     left * m_per + half + qi * q, 6 + qi, right)

        with jax.named_scope("drain_send"):
            a_top_send.wait_send()
            a_bot_send.wait_send()
            for slot in range(8):
                pltpu.make_async_remote_copy(
                    src_ref=c_own.at[pl.ds(0, q), :],
                    dst_ref=out_ref.at[pl.ds(0, q), :],
                    send_sem=c_send_sems.at[slot],
                    recv_sem=c_recv_sems.at[slot],
                    device_id=(right,),
                    device_id_type=pl.DeviceIdType.MESH,
                ).wait_send()
        with jax.named_scope("drain_copy"):
            for cp in local_copies:
                cp.wait()
        inbound = (
            (0, left * m_per + 0 * q),
            (1, left * m_per + 1 * q),
            (2, right * m_per + 2 * q),
            (3, right * m_per + 3 * q),
            (4, diag * m_per + 0 * q),
            (5, diag * m_per + 1 * q),
            (6, diag * m_per + half + 0 * q),
            (7, diag * m_per + half + 1 * q),
        )
        for slot, grow in inbound:
            with jax.named_scope(f"drain_recv#slot={slot}"):
                pltpu.make_async_remote_copy(
                    src_ref=c_own.at[pl.ds(0, q), :],
                    dst_ref=out_ref.at[pl.ds(grow, q), :],
                    send_sem=c_send_sems.at[slot],
                    recv_sem=c_recv_sems.at[slot],
                    device_id=(left,),
                    device_id_type=pl.DeviceIdType.MESH,
                ).wait_recv()

    out_shape = jax.ShapeDtypeStruct((N_DEV * m_per, n), jnp.float32)
    return pl.pallas_call(
        body,
        out_shape=out_shape,
        in_specs=[
            pl.BlockSpec(memory_space=pltpu.VMEM),
            pl.BlockSpec(memory_space=pltpu.VMEM),
        ],
        out_specs=pl.BlockSpec(memory_space=pltpu.MemorySpace.HBM),
        scratch_shapes=[
            pltpu.VMEM((m_per, n), jnp.float32),
            pltpu.VMEM((half, n), jnp.float32),
            pltpu.VMEM((half, n), jnp.float32),
            pltpu.VMEM((half, k), jnp.float32),
            pltpu.VMEM((half, k), jnp.float32),
            pltpu.SemaphoreType.DMA((2,)),
            pltpu.SemaphoreType.DMA((2,)),
            pltpu.SemaphoreType.DMA((8,)),
            pltpu.SemaphoreType.DMA((8,)),
            pltpu.SemaphoreType.DMA((8,)),
        ],
        compiler_params=pltpu.CompilerParams(
            collective_id=0, vmem_limit_bytes=100 * 1024 * 1024
        ),
    )(A, B)


# device time: 204394 ns/iter; 1.3666x vs baseline; 1.3665x over previous
import jax
import jax.numpy as jnp
from jax import lax
from jax.experimental import pallas as pl
from jax.experimental.pallas import tpu as pltpu

N_DEV = 4


def kernel(A, B):
    m_per, k = A.shape
    k2, n = B.shape
    assert k == k2
    q = m_per // 4

    def body(a_ref, b_ref, out_ref, stage, c_send, a_send, a_lb_recv,
             a_rb_recv, c_recv, a_send_sems, a_recv_sems, c_send_sems,
             c_recv_sems, copy_sems):
        my = lax.axis_index("i")
        left = (my + N_DEV - 1) % N_DEV
        right = (my + 1) % N_DEV
        diag = (my + 2) % N_DEV

        with jax.named_scope("barrier"):
            barrier_sem = pltpu.get_barrier_semaphore()
            for nbr in (left, right, diag):
                pl.semaphore_signal(
                    barrier_sem, inc=1,
                    device_id=(nbr,), device_id_type=pl.DeviceIdType.MESH,
                )
            pl.semaphore_wait(barrier_sem, 3)

        with jax.named_scope("a_exchange"):
            a_send[0] = a_ref[pl.ds(3 * q, q), :].astype(jnp.bfloat16)
            a_send[1] = a_ref[pl.ds(2 * q, q), :].astype(jnp.bfloat16)
            a_to_left = pltpu.make_async_remote_copy(
                src_ref=a_send.at[0],
                dst_ref=a_rb_recv,
                send_sem=a_send_sems.at[0],
                recv_sem=a_recv_sems.at[0],
                device_id=(left,),
                device_id_type=pl.DeviceIdType.MESH,
            )
            a_to_left.start()
            a_to_right = pltpu.make_async_remote_copy(
                src_ref=a_send.at[1],
                dst_ref=a_lb_recv,
                send_sem=a_send_sems.at[1],
                recv_sem=a_recv_sems.at[1],
                device_id=(right,),
                device_id_type=pl.DeviceIdType.MESH,
            )
            a_to_right.start()

        local_copies = []

        def compute_chunk(a_chunk, grow, sbuf, sends, scope):
            with jax.named_scope(scope):
                slot_st = sbuf % 2
                if sbuf >= 2:
                    local_copies[sbuf - 2].wait()
                stage[slot_st] = jnp.dot(
                    a_chunk, b_ref[:, :], preferred_element_type=jnp.float32
                )
                cp = pltpu.make_async_copy(
                    stage.at[slot_st], out_ref.at[pl.ds(grow, q), :],
                    copy_sems.at[sbuf],
                )
                cp.start()
                local_copies.append(cp)
                c_send[sbuf] = stage[slot_st].astype(jnp.bfloat16)
                for slot, peer in sends:
                    pltpu.make_async_remote_copy(
                        src_ref=c_send.at[sbuf],
                        dst_ref=c_recv.at[slot],
                        send_sem=c_send_sems.at[slot],
                        recv_sem=c_recv_sems.at[slot],
                        device_id=(peer,),
                        device_id_type=pl.DeviceIdType.MESH,
                    ).start()

        compute_chunk(a_ref[pl.ds(0 * q, q), :], my * m_per + 0 * q, 0,
                      ((0, left), (1, right), (2, diag)), "own#q=0")
        compute_chunk(a_ref[pl.ds(1 * q, q), :], my * m_per + 1 * q, 1,
                      ((3, left), (4, right), (5, diag)), "own#q=1")
        compute_chunk(a_ref[pl.ds(2 * q, q), :], my * m_per + 2 * q, 2,
                      ((6, left),), "own#q=2")
        compute_chunk(a_ref[pl.ds(3 * q, q), :], my * m_per + 3 * q, 3,
                      ((7, right),), "own#q=3")

        with jax.named_scope("wait_a"):
            a_to_right.wait_recv()
            a_to_left.wait_recv()
        compute_chunk(a_lb_recv[:, :].astype(jnp.float32),
                      left * m_per + 2 * q, 4, ((8, right),), "gap_lb_q2")
        compute_chunk(a_rb_recv[:, :].astype(jnp.float32),
                      right * m_per + 3 * q, 5, ((9, left),), "gap_rb_q3")

        inbound = (
            (0, right * m_per + 0 * q),
            (1, left * m_per + 0 * q),
            (3, right * m_per + 1 * q),
            (4, left * m_per + 1 * q),
            (2, diag * m_per + 0 * q),
            (5, diag * m_per + 1 * q),
            (6, right * m_per + 2 * q),
            (7, left * m_per + 3 * q),
            (8, diag * m_per + 2 * q),
            (9, diag * m_per + 3 * q),
        )
        for idx, (slot, grow) in enumerate(inbound):
            with jax.named_scope(f"recv#slot={slot}"):
                pltpu.make_async_remote_copy(
                    src_ref=c_send.at[0],
                    dst_ref=c_recv.at[slot],
                    send_sem=c_send_sems.at[slot],
                    recv_sem=c_recv_sems.at[slot],
                    device_id=(left,),
                    device_id_type=pl.DeviceIdType.MESH,
                ).wait_recv()
                sbuf = 6 + idx
                slot_st = sbuf % 2
                local_copies[sbuf - 2].wait()
                stage[slot_st] = c_recv[slot].astype(jnp.float32)
                cp = pltpu.make_async_copy(
                    stage.at[slot_st], out_ref.at[pl.ds(grow, q), :],
                    copy_sems.at[sbuf],
                )
                cp.start()
                local_copies.append(cp)

        with jax.named_scope("drain"):
            a_to_left.wait_send()
            a_to_right.wait_send()
            for slot in range(10):
                pltpu.make_async_remote_copy(
                    src_ref=c_send.at[0],
                    dst_ref=c_recv.at[slot],
                    send_sem=c_send_sems.at[slot],
                    recv_sem=c_recv_sems.at[slot],
                    device_id=(right,),
                    device_id_type=pl.DeviceIdType.MESH,
                ).wait_send()
            for cp in local_copies[-2:]:
                cp.wait()

    out_shape = jax.ShapeDtypeStruct((N_DEV * m_per, n), jnp.float32)
    return pl.pallas_call(
        body,
        out_shape=out_shape,
        in_specs=[
            pl.BlockSpec(memory_space=pltpu.VMEM),
            pl.BlockSpec(memory_space=pltpu.VMEM),
        ],
        out_specs=pl.BlockSpec(memory_space=pl.ANY),
        scratch_shapes=[
            pltpu.VMEM((2, q, n), jnp.float32),
            pltpu.VMEM((6, q, n), jnp.bfloat16),
            pltpu.VMEM((2, q, k), jnp.bfloat16),
            pltpu.VMEM((q, k), jnp.bfloat16),
            pltpu.VMEM((q, k), jnp.bfloat16),
            pltpu.VMEM((10, q, n), jnp.bfloat16),
            pltpu.SemaphoreType.DMA((2,)),
            pltpu.SemaphoreType.DMA((2,)),
            pltpu.SemaphoreType.DMA((10,)),
            pltpu.SemaphoreType.DMA((10,)),
            pltpu.SemaphoreType.DMA((16,)),
        ],
        compiler_params=pltpu.CompilerParams(
            collective_id=0, vmem_limit_bytes=100 * 1024 * 1024
        ),
    )(A, B)


# device time: 202988 ns/iter; 1.3761x vs baseline; 1.0069x over previous
import jax
import jax.numpy as jnp
from jax import lax
from jax.experimental import pallas as pl
from jax.experimental.pallas import tpu as pltpu

N_DEV = 4


def kernel(A, B):
    m_per, k = A.shape
    k2, n = B.shape
    assert k == k2
    q = m_per // 4

    def body(a_ref, b_ref, out_ref, stage, c_send, a_send, a_lb_recv,
             a_rb_recv, c_recv, a_send_sems, a_recv_sems, c_send_sems,
             c_recv_sems, copy_sems):
        my = lax.axis_index("i")
        left = (my + N_DEV - 1) % N_DEV
        right = (my + 1) % N_DEV
        diag = (my + 2) % N_DEV

        with jax.named_scope("barrier"):
            barrier_sem = pltpu.get_barrier_semaphore()
            for nbr in (left, right, diag):
                pl.semaphore_signal(
                    barrier_sem, inc=1,
                    device_id=(nbr,), device_id_type=pl.DeviceIdType.MESH,
                )
            pl.semaphore_wait(barrier_sem, 3)

        with jax.named_scope("a_exchange"):
            a_send[0] = a_ref[pl.ds(3 * q, q), :].astype(jnp.bfloat16)
            a_send[1] = a_ref[pl.ds(2 * q, q), :].astype(jnp.bfloat16)
            a_to_left = pltpu.make_async_remote_copy(
                src_ref=a_send.at[0],
                dst_ref=a_rb_recv,
                send_sem=a_send_sems.at[0],
                recv_sem=a_recv_sems.at[0],
                device_id=(left,),
                device_id_type=pl.DeviceIdType.MESH,
            )
            a_to_left.start()
            a_to_right = pltpu.make_async_remote_copy(
                src_ref=a_send.at[1],
                dst_ref=a_lb_recv,
                send_sem=a_send_sems.at[1],
                recv_sem=a_recv_sems.at[1],
                device_id=(right,),
                device_id_type=pl.DeviceIdType.MESH,
            )
            a_to_right.start()

        local_copies = []

        def compute_chunk(a_chunk, grow, sbuf, sends, scope):
            with jax.named_scope(scope):
                slot_st = sbuf % 2
                if sbuf >= 2:
                    local_copies[sbuf - 2].wait()
                stage[slot_st] = jnp.dot(
                    a_chunk, b_ref[:, :], preferred_element_type=jnp.float32
                )
                cp = pltpu.make_async_copy(
                    stage.at[slot_st], out_ref.at[pl.ds(grow, q), :],
                    copy_sems.at[sbuf],
                )
                cp.start()
                local_copies.append(cp)
                c_send[sbuf] = stage[slot_st].astype(jnp.bfloat16)
                for slot, peer in sends:
                    pltpu.make_async_remote_copy(
                        src_ref=c_send.at[sbuf],
                        dst_ref=c_recv.at[slot],
                        send_sem=c_send_sems.at[slot],
                        recv_sem=c_recv_sems.at[slot],
                        device_id=(peer,),
                        device_id_type=pl.DeviceIdType.MESH,
                    ).start()

        compute_chunk(a_ref[pl.ds(0 * q, q), :], my * m_per + 0 * q, 0,
                      ((0, left), (1, right), (2, diag)), "own#q=0")
        compute_chunk(a_ref[pl.ds(1 * q, q), :], my * m_per + 1 * q, 1,
                      ((3, left), (4, right), (5, diag)), "own#q=1")
        compute_chunk(a_ref[pl.ds(2 * q, q), :], my * m_per + 2 * q, 2,
                      ((6, left),), "own#q=2")
        compute_chunk(a_ref[pl.ds(3 * q, q), :], my * m_per + 3 * q, 3,
                      ((7, right),), "own#q=3")

        hq = q // 2
        with jax.named_scope("wait_a"):
            a_to_right.wait_recv()
            a_to_left.wait_recv()
        gaps = ((a_lb_recv, 4, left * m_per + 2 * q, 8, right),
                (a_rb_recv, 5, right * m_per + 3 * q, 9, left))
        for h in range(2):
            for a_recv, sbuf, gbase, slot0, peer in gaps:
                slot = slot0 + 2 * h
                with jax.named_scope(f"gap#slot={slot}"):
                    u = len(local_copies)
                    slot_st = u % 2
                    local_copies[u - 2].wait()
                    stage[slot_st, pl.ds(0, hq), :] = jnp.dot(
                        a_recv[pl.ds(h * hq, hq), :].astype(jnp.float32),
                        b_ref[:, :], preferred_element_type=jnp.float32,
                    )
                    cp = pltpu.make_async_copy(
                        stage.at[slot_st, pl.ds(0, hq), :],
                        out_ref.at[pl.ds(gbase + h * hq, hq), :],
                        copy_sems.at[u],
                    )
                    cp.start()
                    local_copies.append(cp)
                    c_send[sbuf, pl.ds(h * hq, hq), :] = (
                        stage[slot_st, pl.ds(0, hq), :].astype(jnp.bfloat16)
                    )
                    pltpu.make_async_remote_copy(
                        src_ref=c_send.at[sbuf, pl.ds(h * hq, hq), :],
                        dst_ref=c_recv.at[slot0, pl.ds(h * hq, hq), :],
                        send_sem=c_send_sems.at[slot],
                        recv_sem=c_recv_sems.at[slot],
                        device_id=(peer,),
                        device_id_type=pl.DeviceIdType.MESH,
                    ).start()

        inbound = (
            (0, 0, 0, q, right * m_per + 0 * q),
            (1, 1, 0, q, left * m_per + 0 * q),
            (3, 3, 0, q, right * m_per + 1 * q),
            (4, 4, 0, q, left * m_per + 1 * q),
            (2, 2, 0, q, diag * m_per + 0 * q),
            (5, 5, 0, q, diag * m_per + 1 * q),
            (6, 6, 0, q, right * m_per + 2 * q),
            (7, 7, 0, q, left * m_per + 3 * q),
            (8, 8, 0, hq, diag * m_per + 2 * q),
            (9, 9, 0, hq, diag * m_per + 3 * q),
            (10, 8, hq, hq, diag * m_per + 2 * q + hq),
            (11, 9, hq, hq, diag * m_per + 3 * q + hq),
        )
        for slot, rbuf, roff, rows, grow in inbound:
            with jax.named_scope(f"recv#slot={slot}"):
                pltpu.make_async_remote_copy(
                    src_ref=c_send.at[0, pl.ds(0, rows), :],
                    dst_ref=c_recv.at[rbuf, pl.ds(roff, rows), :],
                    send_sem=c_send_sems.at[slot],
                    recv_sem=c_recv_sems.at[slot],
                    device_id=(left,),
                    device_id_type=pl.DeviceIdType.MESH,
                ).wait_recv()
                u = len(local_copies)
                slot_st = u % 2
                local_copies[u - 2].wait()
                stage[slot_st, pl.ds(0, rows), :] = (
                    c_recv[rbuf, pl.ds(roff, rows), :].astype(jnp.float32)
                )
                cp = pltpu.make_async_copy(
                    stage.at[slot_st, pl.ds(0, rows), :],
                    out_ref.at[pl.ds(grow, rows), :],
                    copy_sems.at[u],
                )
                cp.start()
                local_copies.append(cp)

        with jax.named_scope("drain"):
            a_to_left.wait_send()
            a_to_right.wait_send()
            for slot in range(12):
                rows = q if slot < 8 else hq
                pltpu.make_async_remote_copy(
                    src_ref=c_send.at[0, pl.ds(0, rows), :],
                    dst_ref=c_recv.at[0, pl.ds(0, rows), :],
                    send_sem=c_send_sems.at[slot],
                    recv_sem=c_recv_sems.at[slot],
                    device_id=(right,),
                    device_id_type=pl.DeviceIdType.MESH,
                ).wait_send()
            for cp in local_copies[-2:]:
                cp.wait()

    out_shape = jax.ShapeDtypeStruct((N_DEV * m_per, n), jnp.float32)
    return pl.pallas_call(
        body,
        out_shape=out_shape,
        in_specs=[
            pl.BlockSpec(memory_space=pltpu.VMEM),
            pl.BlockSpec(memory_space=pltpu.VMEM),
        ],
        out_specs=pl.BlockSpec(memory_space=pl.ANY),
        scratch_shapes=[
            pltpu.VMEM((2, q, n), jnp.float32),
            pltpu.VMEM((6, q, n), jnp.bfloat16),
            pltpu.VMEM((2, q, k), jnp.bfloat16),
            pltpu.VMEM((q, k), jnp.bfloat16),
            pltpu.VMEM((q, k), jnp.bfloat16),
            pltpu.VMEM((10, q, n), jnp.bfloat16),
            pltpu.SemaphoreType.DMA((2,)),
            pltpu.SemaphoreType.DMA((2,)),
            pltpu.SemaphoreType.DMA((12,)),
            pltpu.SemaphoreType.DMA((12,)),
            pltpu.SemaphoreType.DMA((20,)),
        ],
        compiler_params=pltpu.CompilerParams(
            collective_id=0, vmem_limit_bytes=100 * 1024 * 1024
        ),
    )(A, B)
